# Initial kernel scaffold; baseline (speedup 1.0000x reference)
import jax
import jax.numpy as jnp
from jax.experimental import pallas as pl


def kernel(x, edge_index, W1, b1, Wq0, bq0, Wk0, bk0, Wv0, bv0, Wq1, bq1, Wk1, bk1, Wv1, bv1, W2, b2):
    raise NotImplementedError("write your pallas kernel here")


# speedup vs baseline: 1.3351x; 1.3351x over previous
import functools

import jax
import jax.numpy as jnp
from jax import lax
from jax.experimental import pallas as pl
from jax.experimental.pallas import tpu as pltpu
from jax.experimental.pallas import tpu_sc as plsc

NCORE = 2
NSUB = 16
NW = NCORE * NSUB
CH = 128
HEADS = 8
DH = 16
NPAD = 10240
BN = 512
BE = 64
RPS = NPAD // NSUB


def _block_diag(W):
    G, ci, co = W.shape
    out = jnp.zeros((G * ci, G * co), W.dtype)
    for g in range(G):
        out = out.at[g * ci:(g + 1) * ci, g * co:(g + 1) * co].set(W[g])
    return out


def _degree(col_p):
    ept = col_p.shape[0] // NW
    nch = ept // BE
    mesh = plsc.VectorSubcoreMesh(core_axis_name="c", subcore_axis_name="s")

    @functools.partial(
        pl.kernel,
        out_type=jax.ShapeDtypeStruct((NCORE, NPAD, 16), jnp.float32),
        mesh=mesh,
        compiler_params=pltpu.CompilerParams(needs_layout_passes=False,
                                             use_tc_tiling_on_sc=False),
        scratch_types=[
            pltpu.VMEM((BE,), jnp.int32),
            pltpu.VMEM((BE, 16), jnp.float32),
            pltpu.VMEM((RPS, 16), jnp.float32),
            pltpu.MemorySpace.VMEM_SHARED((NPAD, 16), jnp.float32),
        ],
    )
    def k(col_hbm, deg_hbm, col_v, ones_v, zb_v, acc_s):
        cid = lax.axis_index("c")
        sid = lax.axis_index("s")
        wid = sid * NCORE + cid
        zeros16 = jnp.zeros((16,), jnp.float32)
        ones16 = jnp.ones((16,), jnp.float32)

        def initz(i, _):
            zb_v[i, :] = zeros16
            return 0

        lax.fori_loop(0, RPS, initz, 0)

        def inito(i, _):
            ones_v[i, :] = ones16
            return 0

        lax.fori_loop(0, BE, inito, 0)
        pltpu.sync_copy(zb_v, acc_s.at[pl.ds(sid * RPS, RPS)])
        plsc.subcore_barrier()

        def chunk(j, _):
            base = wid * ept + j * BE
            pltpu.sync_copy(col_hbm.at[pl.ds(base, BE)], col_v)
            pltpu.sync_copy(ones_v, acc_s.at[col_v], add=True)
            return 0

        lax.fori_loop(0, nch, chunk, 0)
        plsc.subcore_barrier()
        pltpu.sync_copy(acc_s.at[pl.ds(sid * RPS, RPS)], zb_v)
        pltpu.sync_copy(zb_v, deg_hbm.at[cid, pl.ds(sid * RPS, RPS)])

    return k(col_p)


def _conv(L, row_p, col_p, Q, Kc, Vc):
    ept = row_p.shape[0] // NW
    nch = ept // BE
    CK = L * CH
    mesh = plsc.VectorSubcoreMesh(core_axis_name="c", subcore_axis_name="s")

    @functools.partial(
        pl.kernel,
        out_type=jax.ShapeDtypeStruct((NCORE, NPAD, CH), jnp.float32),
        mesh=mesh,
        compiler_params=pltpu.CompilerParams(needs_layout_passes=False,
                                             use_tc_tiling_on_sc=False),
        scratch_types=[
            pltpu.VMEM((BE,), jnp.int32),
            pltpu.VMEM((BE,), jnp.int32),
            pltpu.VMEM((BE, CH), jnp.float32),
            pltpu.VMEM((BE, CK), jnp.float32),
            pltpu.VMEM((BE, CK), jnp.float32),
            pltpu.MemorySpace.VMEM_SHARED((NPAD, CH), jnp.float32),
            pltpu.SemaphoreType.DMA,
        ],
    )
    def k(row_hbm, col_hbm, q_hbm, k_hbm, v_hbm, out_hbm,
          row_v, col_v, q_v, k_v, v_v, acc_s, sem):
        cid = lax.axis_index("c")
        sid = lax.axis_index("s")
        wid = sid * NCORE + cid
        zeros16 = jnp.zeros((16,), jnp.float32)

        def initz(i, _):
            q_v[i // HEADS, pl.ds((i % HEADS) * DH, DH)] = zeros16
            return 0

        lax.fori_loop(0, BE * HEADS, initz, 0)
        for t in range(RPS // BE):
            pltpu.sync_copy(q_v, acc_s.at[pl.ds(sid * RPS + t * BE, BE)])
        plsc.subcore_barrier()

        def chunk(j, _):
            base = wid * ept + j * BE
            pltpu.sync_copy(row_hbm.at[pl.ds(base, BE)], row_v)
            pltpu.sync_copy(col_hbm.at[pl.ds(base, BE)], col_v)
            cp_q = pltpu.async_copy(q_hbm.at[col_v], q_v, sem)
            cp_k = pltpu.async_copy(k_hbm.at[row_v], k_v, sem)
            cp_v = pltpu.async_copy(v_hbm.at[row_v], v_v, sem)
            cp_q.wait()
            cp_k.wait()
            cp_v.wait()

            def edge16(g, _):
                rvec = g * 16 + lax.iota(jnp.int32, 16)

                def cv(c):
                    return jnp.full((16,), c, jnp.int32)

                for h in range(HEADS):
                    s = [jnp.zeros((16,), jnp.float32) for _ in range(L)]
                    for d in range(DH):
                        c = h * DH + d
                        qd = plsc.load_gather(q_v, [rvec, cv(c)])
                        for l in range(L):
                            kd = plsc.load_gather(k_v, [rvec, cv(l * CH + c)])
                            s[l] = s[l] + qd * kd
                    if L == 1:
                        m = jnp.maximum(s[0] * 0.25, 0.0)
                        e0 = jnp.exp(s[0] * 0.25 - m)
                        a = [e0 / (e0 + jnp.exp(-m))]
                    else:
                        s0 = s[0] * 0.25
                        s1 = s[1] * 0.25
                        m = jnp.maximum(jnp.maximum(s0, s1), 0.0)
                        e0 = jnp.exp(s0 - m)
                        e1 = jnp.exp(s1 - m)
                        den = e0 + e1 + jnp.exp(-m)
                        a = [e0 / den, e1 / den]
                    for d in range(DH):
                        c = h * DH + d
                        acc = a[0] * plsc.load_gather(v_v, [rvec, cv(c)])
                        if L == 2:
                            acc = acc + a[1] * plsc.load_gather(
                                v_v, [rvec, cv(CH + c)])
                        plsc.store_scatter(q_v, [rvec, cv(c)], acc)
                return 0

            lax.fori_loop(0, BE // 16, edge16, 0)
            pltpu.sync_copy(q_v, acc_s.at[col_v], add=True)
            return 0

        lax.fori_loop(0, nch, chunk, 0)
        plsc.subcore_barrier()
        for t in range(RPS // BE):
            r0 = sid * RPS + t * BE
            pltpu.sync_copy(acc_s.at[pl.ds(r0, BE)], q_v)
            pltpu.sync_copy(q_v, out_hbm.at[cid, pl.ds(r0, BE)])

    return k(row_p, col_p, Q, Kc, Vc)


def _tc1(x_p, W1, b1, Wq, bq, Wk, bk, Wv, bv, deg2):
    grid = (NPAD // BN,)
    nreal_ref = None

    def body(x_ref, w1_ref, b1_ref, wq_ref, bq_ref, wk_ref, bk_ref,
             wv_ref, bv_ref, deg_ref, h_ref, q_ref, k_ref, v_ref, dis_ref):
        i = pl.program_id(0)
        h = jnp.maximum(x_ref[...] @ w1_ref[...] + b1_ref[...], 0.0)
        deg = deg_ref[0, :, 0:1] + deg_ref[1, :, 0:1] + 1.0
        rows = i * BN + lax.broadcasted_iota(jnp.int32, (BN, 1), 0)
        dis = jnp.where(rows < _NREAL, lax.rsqrt(deg), 0.0)
        h_ref[...] = h
        q_ref[...] = h @ wq_ref[...] + bq_ref[...]
        k_ref[...] = h @ wk_ref[...] + bk_ref[...]
        v_ref[...] = (h @ wv_ref[...] + bv_ref[...]) * dis
        dis_ref[...] = jnp.broadcast_to(dis, (BN, CH))

    full = lambda s: pl.BlockSpec(s, lambda i: (0,) * len(s))
    rowb = pl.BlockSpec((BN, CH), lambda i: (i, 0))
    return pl.pallas_call(
        body,
        grid=grid,
        in_specs=[rowb, full((CH, CH)), full((1, CH)), full((CH, CH)),
                  full((1, CH)), full((CH, CH)), full((1, CH)),
                  full((CH, CH)), full((1, CH)),
                  pl.BlockSpec((2, BN, 16), lambda i: (0, i, 0))],
        out_specs=[rowb, rowb, rowb, rowb, rowb],
        out_shape=[jax.ShapeDtypeStruct((NPAD, CH), jnp.float32)] * 5,
    )(x_p, W1, b1, Wq, bq, Wk, bk, Wv, bv, deg2)


def _tc2(h, agg0, dis, Wq, bq, Wk, bk, Wv, bv):
    grid = (NPAD // BN,)

    def body(h_ref, a_ref, dis_ref, wq_ref, bq_ref, wk_ref, bk_ref,
             wv_ref, bv_ref, q_ref, k_ref, v_ref):
        dis = dis_ref[...]
        h = h_ref[...]
        h1 = jnp.maximum(dis * (a_ref[0] + a_ref[1]), 0.0)
        q_ref[...] = h1 @ wq_ref[...] + bq_ref[...]
        k_ref[:, 0:CH] = h @ wk_ref[...] + bk_ref[...]
        k_ref[:, CH:2 * CH] = h1 @ wk_ref[...] + bk_ref[...]
        v_ref[:, 0:CH] = (h @ wv_ref[...] + bv_ref[...]) * dis
        v_ref[:, CH:2 * CH] = (h1 @ wv_ref[...] + bv_ref[...]) * dis

    full = lambda s: pl.BlockSpec(s, lambda i: (0,) * len(s))
    rowb = pl.BlockSpec((BN, CH), lambda i: (i, 0))
    rowb2 = pl.BlockSpec((BN, 2 * CH), lambda i: (i, 0))
    return pl.pallas_call(
        body,
        grid=grid,
        in_specs=[rowb, pl.BlockSpec((2, BN, CH), lambda i: (0, i, 0)), rowb,
                  full((CH, CH)), full((1, CH)), full((CH, CH)), full((1, CH)),
                  full((CH, CH)), full((1, CH))],
        out_specs=[rowb, rowb2, rowb2],
        out_shape=[jax.ShapeDtypeStruct((NPAD, CH), jnp.float32),
                   jax.ShapeDtypeStruct((NPAD, 2 * CH), jnp.float32),
                   jax.ShapeDtypeStruct((NPAD, 2 * CH), jnp.float32)],
    )(h, agg0, dis, Wq, bq, Wk, bk, Wv, bv)


def _tc3(agg1, dis, W2p, b2p, nclass):
    grid = (NPAD // BN,)

    def body(a_ref, dis_ref, w2_ref, b2_ref, o_ref):
        h2 = jnp.maximum(dis_ref[...] * (a_ref[0] + a_ref[1]), 0.0)
        lg = h2 @ w2_ref[...] + b2_ref[...]
        colm = lax.broadcasted_iota(jnp.int32, (BN, CH), 1) < nclass
        lgm = jnp.where(colm, lg, -1e30)
        mx = jnp.max(lgm, axis=1, keepdims=True)
        sm = jnp.sum(jnp.exp(lgm - mx), axis=1, keepdims=True)
        o_ref[...] = lg - mx - jnp.log(sm)

    full = lambda s: pl.BlockSpec(s, lambda i: (0,) * len(s))
    rowb = pl.BlockSpec((BN, CH), lambda i: (i, 0))
    return pl.pallas_call(
        body,
        grid=grid,
        in_specs=[pl.BlockSpec((2, BN, CH), lambda i: (0, i, 0)), rowb,
                  full((CH, CH)), full((1, CH))],
        out_specs=rowb,
        out_shape=jax.ShapeDtypeStruct((NPAD, CH), jnp.float32),
    )(agg1, dis, W2p, b2p)


_NREAL = 10000


def kernel(x, edge_index, W1, b1, Wq0, bq0, Wk0, bk0, Wv0, bv0,
           Wq1, bq1, Wk1, bk1, Wv1, bv1, W2, b2):
    global _NREAL
    N, F = x.shape
    _NREAL = N
    E = edge_index.shape[1]
    nclass = W2.shape[1]
    idt = edge_index.dtype

    loops = jnp.arange(N, dtype=idt)
    row = jnp.concatenate([edge_index[0], loops])
    col = jnp.concatenate([edge_index[1], loops])
    etot = E + N
    epad = ((etot + NW * BE - 1) // (NW * BE)) * (NW * BE)
    padidx = jnp.full((epad - etot,), NPAD - 1, idt)
    row_p = jnp.concatenate([row, padidx]).astype(jnp.int32)
    col_p = jnp.concatenate([col, padidx]).astype(jnp.int32)

    x_p = jnp.pad(x, ((0, NPAD - N), (0, 0)))
    r2 = lambda b: b.reshape(1, CH)
    bd = _block_diag

    deg2 = _degree(col_p)
    h, q0, k0, v0p, dis = _tc1(x_p, W1, r2(b1), bd(Wq0), r2(bq0),
                               bd(Wk0), r2(bk0), bd(Wv0), r2(bv0), deg2)
    agg0 = _conv(1, row_p, col_p, q0, k0, v0p)
    q1, k1c, v1c = _tc2(h, agg0, dis, bd(Wq1), r2(bq1), bd(Wk1), r2(bk1),
                        bd(Wv1), r2(bv1))
    agg1 = _conv(2, row_p, col_p, q1, k1c, v1c)
    W2p = jnp.pad(W2, ((0, 0), (0, CH - nclass)))
    b2p = jnp.pad(b2, (0, CH - nclass)).reshape(1, CH)
    out = _tc3(agg1, dis, W2p, b2p, nclass)
    return out[:N, :nclass]


# speedup vs baseline: 2.1679x; 1.6238x over previous
import functools

import jax
import jax.numpy as jnp
from jax import lax
from jax.experimental import pallas as pl
from jax.experimental.pallas import tpu as pltpu
from jax.experimental.pallas import tpu_sc as plsc

NCORE = 2
NSUB = 16
NW = NCORE * NSUB
CH = 128
HEADS = 8
DH = 16
NPAD = 10240
BN = 512
BE = 64
RPS = NPAD // NSUB


def _block_diag(W):
    G, ci, co = W.shape
    out = jnp.zeros((G * ci, G * co), W.dtype)
    for g in range(G):
        out = out.at[g * ci:(g + 1) * ci, g * co:(g + 1) * co].set(W[g])
    return out


def _degree(col_p):
    ept = col_p.shape[0] // NW
    nch = ept // BE
    mesh = plsc.VectorSubcoreMesh(core_axis_name="c", subcore_axis_name="s")

    @functools.partial(
        pl.kernel,
        out_type=jax.ShapeDtypeStruct((NCORE, NPAD, 16), jnp.float32),
        mesh=mesh,
        compiler_params=pltpu.CompilerParams(needs_layout_passes=False,
                                             use_tc_tiling_on_sc=False),
        scratch_types=[
            pltpu.VMEM((BE,), jnp.int32),
            pltpu.VMEM((BE, 16), jnp.float32),
            pltpu.VMEM((RPS, 16), jnp.float32),
            pltpu.MemorySpace.VMEM_SHARED((NPAD, 16), jnp.float32),
        ],
    )
    def k(col_hbm, deg_hbm, col_v, ones_v, zb_v, acc_s):
        cid = lax.axis_index("c")
        sid = lax.axis_index("s")
        wid = sid * NCORE + cid
        zeros16 = jnp.zeros((16,), jnp.float32)
        ones16 = jnp.ones((16,), jnp.float32)

        def initz(i, _):
            zb_v[i, :] = zeros16
            return 0

        lax.fori_loop(0, RPS, initz, 0)

        def inito(i, _):
            ones_v[i, :] = ones16
            return 0

        lax.fori_loop(0, BE, inito, 0)
        pltpu.sync_copy(zb_v, acc_s.at[pl.ds(sid * RPS, RPS)])
        plsc.subcore_barrier()

        def chunk(j, _):
            base = wid * ept + j * BE
            pltpu.sync_copy(col_hbm.at[pl.ds(base, BE)], col_v)
            pltpu.sync_copy(ones_v, acc_s.at[col_v], add=True)
            return 0

        lax.fori_loop(0, nch, chunk, 0)
        plsc.subcore_barrier()
        pltpu.sync_copy(acc_s.at[pl.ds(sid * RPS, RPS)], zb_v)
        pltpu.sync_copy(zb_v, deg_hbm.at[cid, pl.ds(sid * RPS, RPS)])

    return k(col_p)


def _conv(L, row_p, col_p, Q, Kc, Vc):
    ept = row_p.shape[0] // NW
    nch = ept // BE
    CK = L * CH
    mesh = plsc.VectorSubcoreMesh(core_axis_name="c", subcore_axis_name="s")

    @functools.partial(
        pl.kernel,
        out_type=jax.ShapeDtypeStruct((NCORE, NPAD, CH), jnp.float32),
        mesh=mesh,
        compiler_params=pltpu.CompilerParams(needs_layout_passes=False,
                                             use_tc_tiling_on_sc=False),
        scratch_types=[
            pltpu.VMEM((BE,), jnp.int32),
            pltpu.VMEM((BE,), jnp.int32),
            pltpu.VMEM((BE, CH), jnp.float32),
            pltpu.VMEM((BE, CK), jnp.float32),
            pltpu.VMEM((BE, CK), jnp.float32),
            pltpu.MemorySpace.VMEM_SHARED((NPAD, CH), jnp.float32),
            pltpu.SemaphoreType.DMA,
        ],
    )
    def k(row_hbm, col_hbm, q_hbm, k_hbm, v_hbm, out_hbm,
          row_v, col_v, q_v, k_v, v_v, acc_s, sem):
        cid = lax.axis_index("c")
        sid = lax.axis_index("s")
        wid = sid * NCORE + cid
        zeros16 = jnp.zeros((16,), jnp.float32)

        def initz(i, _):
            q_v[i // HEADS, pl.ds((i % HEADS) * DH, DH)] = zeros16
            return 0

        lax.fori_loop(0, BE * HEADS, initz, 0)
        for t in range(RPS // BE):
            pltpu.sync_copy(q_v, acc_s.at[pl.ds(sid * RPS + t * BE, BE)])
        plsc.subcore_barrier()

        def chunk(j, _):
            base = wid * ept + j * BE
            pltpu.sync_copy(row_hbm.at[pl.ds(base, BE)], row_v)
            pltpu.sync_copy(col_hbm.at[pl.ds(base, BE)], col_v)
            cp_q = pltpu.async_copy(q_hbm.at[col_v], q_v, sem)
            cp_k = pltpu.async_copy(k_hbm.at[row_v], k_v, sem)
            cp_v = pltpu.async_copy(v_hbm.at[row_v], v_v, sem)
            cp_q.wait()
            cp_k.wait()
            cp_v.wait()

            def edge(e, _):
                for h in range(HEADS):
                    o = h * DH
                    q = q_v[e, pl.ds(o, DH)]
                    if L == 1:
                        k0 = k_v[e, pl.ds(o, DH)]
                        s0 = jnp.sum(q * k0) * 0.25
                        m = jnp.maximum(s0, 0.0)
                        e0 = jnp.exp(jnp.full((DH,), s0 - m, jnp.float32))
                        em = jnp.exp(jnp.full((DH,), -m, jnp.float32))
                        mh = (e0 / (e0 + em)) * v_v[e, pl.ds(o, DH)]
                    else:
                        k0 = k_v[e, pl.ds(o, DH)]
                        k1 = k_v[e, pl.ds(CH + o, DH)]
                        s0 = jnp.sum(q * k0) * 0.25
                        s1 = jnp.sum(q * k1) * 0.25
                        m = jnp.maximum(jnp.maximum(s0, s1), 0.0)
                        e0 = jnp.exp(jnp.full((DH,), s0 - m, jnp.float32))
                        e1 = jnp.exp(jnp.full((DH,), s1 - m, jnp.float32))
                        em = jnp.exp(jnp.full((DH,), -m, jnp.float32))
                        v0 = v_v[e, pl.ds(o, DH)]
                        v1 = v_v[e, pl.ds(CH + o, DH)]
                        mh = (e0 * v0 + e1 * v1) / (e0 + e1 + em)
                    q_v[e, pl.ds(o, DH)] = mh
                return 0

            lax.fori_loop(0, BE, edge, 0)
            pltpu.sync_copy(q_v, acc_s.at[col_v], add=True)
            return 0

        lax.fori_loop(0, nch, chunk, 0)
        plsc.subcore_barrier()
        for t in range(RPS // BE):
            r0 = sid * RPS + t * BE
            pltpu.sync_copy(acc_s.at[pl.ds(r0, BE)], q_v)
            pltpu.sync_copy(q_v, out_hbm.at[cid, pl.ds(r0, BE)])

    return k(row_p, col_p, Q, Kc, Vc)


def _tc1(x_p, W1, b1, Wq, bq, Wk, bk, Wv, bv, deg2):
    grid = (NPAD // BN,)
    nreal_ref = None

    def body(x_ref, w1_ref, b1_ref, wq_ref, bq_ref, wk_ref, bk_ref,
             wv_ref, bv_ref, deg_ref, h_ref, q_ref, k_ref, v_ref, dis_ref):
        i = pl.program_id(0)
        h = jnp.maximum(x_ref[...] @ w1_ref[...] + b1_ref[...], 0.0)
        deg = deg_ref[0, :, 0:1] + deg_ref[1, :, 0:1] + 1.0
        rows = i * BN + lax.broadcasted_iota(jnp.int32, (BN, 1), 0)
        dis = jnp.where(rows < _NREAL, lax.rsqrt(deg), 0.0)
        h_ref[...] = h
        q_ref[...] = h @ wq_ref[...] + bq_ref[...]
        k_ref[...] = h @ wk_ref[...] + bk_ref[...]
        v_ref[...] = (h @ wv_ref[...] + bv_ref[...]) * dis
        dis_ref[...] = jnp.broadcast_to(dis, (BN, CH))

    full = lambda s: pl.BlockSpec(s, lambda i: (0,) * len(s))
    rowb = pl.BlockSpec((BN, CH), lambda i: (i, 0))
    return pl.pallas_call(
        body,
        grid=grid,
        in_specs=[rowb, full((CH, CH)), full((1, CH)), full((CH, CH)),
                  full((1, CH)), full((CH, CH)), full((1, CH)),
                  full((CH, CH)), full((1, CH)),
                  pl.BlockSpec((2, BN, 16), lambda i: (0, i, 0))],
        out_specs=[rowb, rowb, rowb, rowb, rowb],
        out_shape=[jax.ShapeDtypeStruct((NPAD, CH), jnp.float32)] * 5,
    )(x_p, W1, b1, Wq, bq, Wk, bk, Wv, bv, deg2)


def _tc2(h, agg0, dis, Wq, bq, Wk, bk, Wv, bv):
    grid = (NPAD // BN,)

    def body(h_ref, a_ref, dis_ref, wq_ref, bq_ref, wk_ref, bk_ref,
             wv_ref, bv_ref, q_ref, k_ref, v_ref):
        dis = dis_ref[...]
        h = h_ref[...]
        h1 = jnp.maximum(dis * (a_ref[0] + a_ref[1]), 0.0)
        q_ref[...] = h1 @ wq_ref[...] + bq_ref[...]
        k_ref[:, 0:CH] = h @ wk_ref[...] + bk_ref[...]
        k_ref[:, CH:2 * CH] = h1 @ wk_ref[...] + bk_ref[...]
        v_ref[:, 0:CH] = (h @ wv_ref[...] + bv_ref[...]) * dis
        v_ref[:, CH:2 * CH] = (h1 @ wv_ref[...] + bv_ref[...]) * dis

    full = lambda s: pl.BlockSpec(s, lambda i: (0,) * len(s))
    rowb = pl.BlockSpec((BN, CH), lambda i: (i, 0))
    rowb2 = pl.BlockSpec((BN, 2 * CH), lambda i: (i, 0))
    return pl.pallas_call(
        body,
        grid=grid,
        in_specs=[rowb, pl.BlockSpec((2, BN, CH), lambda i: (0, i, 0)), rowb,
                  full((CH, CH)), full((1, CH)), full((CH, CH)), full((1, CH)),
                  full((CH, CH)), full((1, CH))],
        out_specs=[rowb, rowb2, rowb2],
        out_shape=[jax.ShapeDtypeStruct((NPAD, CH), jnp.float32),
                   jax.ShapeDtypeStruct((NPAD, 2 * CH), jnp.float32),
                   jax.ShapeDtypeStruct((NPAD, 2 * CH), jnp.float32)],
    )(h, agg0, dis, Wq, bq, Wk, bk, Wv, bv)


def _tc3(agg1, dis, W2p, b2p, nclass):
    grid = (NPAD // BN,)

    def body(a_ref, dis_ref, w2_ref, b2_ref, o_ref):
        h2 = jnp.maximum(dis_ref[...] * (a_ref[0] + a_ref[1]), 0.0)
        lg = h2 @ w2_ref[...] + b2_ref[...]
        colm = lax.broadcasted_iota(jnp.int32, (BN, CH), 1) < nclass
        lgm = jnp.where(colm, lg, -1e30)
        mx = jnp.max(lgm, axis=1, keepdims=True)
        sm = jnp.sum(jnp.exp(lgm - mx), axis=1, keepdims=True)
        o_ref[...] = lg - mx - jnp.log(sm)

    full = lambda s: pl.BlockSpec(s, lambda i: (0,) * len(s))
    rowb = pl.BlockSpec((BN, CH), lambda i: (i, 0))
    return pl.pallas_call(
        body,
        grid=grid,
        in_specs=[pl.BlockSpec((2, BN, CH), lambda i: (0, i, 0)), rowb,
                  full((CH, CH)), full((1, CH))],
        out_specs=rowb,
        out_shape=jax.ShapeDtypeStruct((NPAD, CH), jnp.float32),
    )(agg1, dis, W2p, b2p)


_NREAL = 10000


def kernel(x, edge_index, W1, b1, Wq0, bq0, Wk0, bk0, Wv0, bv0,
           Wq1, bq1, Wk1, bk1, Wv1, bv1, W2, b2):
    global _NREAL
    N, F = x.shape
    _NREAL = N
    E = edge_index.shape[1]
    nclass = W2.shape[1]
    idt = edge_index.dtype

    loops = jnp.arange(N, dtype=idt)
    row = jnp.concatenate([edge_index[0], loops])
    col = jnp.concatenate([edge_index[1], loops])
    etot = E + N
    epad = ((etot + NW * BE - 1) // (NW * BE)) * (NW * BE)
    padidx = jnp.full((epad - etot,), NPAD - 1, idt)
    row_p = jnp.concatenate([row, padidx]).astype(jnp.int32)
    col_p = jnp.concatenate([col, padidx]).astype(jnp.int32)

    x_p = jnp.pad(x, ((0, NPAD - N), (0, 0)))
    r2 = lambda b: b.reshape(1, CH)
    bd = _block_diag

    deg2 = _degree(col_p)
    h, q0, k0, v0p, dis = _tc1(x_p, W1, r2(b1), bd(Wq0), r2(bq0),
                               bd(Wk0), r2(bk0), bd(Wv0), r2(bv0), deg2)
    agg0 = _conv(1, row_p, col_p, q0, k0, v0p)
    q1, k1c, v1c = _tc2(h, agg0, dis, bd(Wq1), r2(bq1), bd(Wk1), r2(bk1),
                        bd(Wv1), r2(bv1))
    agg1 = _conv(2, row_p, col_p, q1, k1c, v1c)
    W2p = jnp.pad(W2, ((0, 0), (0, CH - nclass)))
    b2p = jnp.pad(b2, (0, CH - nclass)).reshape(1, CH)
    out = _tc3(agg1, dis, W2p, b2p, nclass)
    return out[:N, :nclass]


# speedup vs baseline: 2.4609x; 1.1351x over previous
import functools

import jax
import jax.numpy as jnp
from jax import lax
from jax.experimental import pallas as pl
from jax.experimental.pallas import tpu as pltpu
from jax.experimental.pallas import tpu_sc as plsc

NCORE = 2
NSUB = 16
NW = NCORE * NSUB
CH = 128
HEADS = 8
DH = 16
NPAD = 10240
BN = 512
BE = 64
RPS = NPAD // NSUB

_GDN = lax.GatherDimensionNumbers(
    offset_dims=(), collapsed_slice_dims=(0,), start_index_map=(0,))


def _dot16(a, b):
    p = a * b
    for k in (8, 4, 2, 1):
        perm = (jnp.arange(16, dtype=jnp.int32) ^ k)[:, None]
        p = p + lax.gather(p, perm, _GDN, (1,),
                           mode=lax.GatherScatterMode.PROMISE_IN_BOUNDS)
    return p


def _block_diag(W):
    G, ci, co = W.shape
    out = jnp.zeros((G * ci, G * co), W.dtype)
    for g in range(G):
        out = out.at[g * ci:(g + 1) * ci, g * co:(g + 1) * co].set(W[g])
    return out


def _degree(col_p):
    ept = col_p.shape[0] // NW
    nch = ept // BE
    mesh = plsc.VectorSubcoreMesh(core_axis_name="c", subcore_axis_name="s")

    @functools.partial(
        pl.kernel,
        out_type=jax.ShapeDtypeStruct((NCORE, NPAD, 16), jnp.float32),
        mesh=mesh,
        compiler_params=pltpu.CompilerParams(needs_layout_passes=False,
                                             use_tc_tiling_on_sc=False),
        scratch_types=[
            pltpu.VMEM((BE,), jnp.int32),
            pltpu.VMEM((BE, 16), jnp.float32),
            pltpu.VMEM((RPS, 16), jnp.float32),
            pltpu.MemorySpace.VMEM_SHARED((NPAD, 16), jnp.float32),
        ],
    )
    def k(col_hbm, deg_hbm, col_v, ones_v, zb_v, acc_s):
        cid = lax.axis_index("c")
        sid = lax.axis_index("s")
        wid = sid * NCORE + cid
        zeros16 = jnp.zeros((16,), jnp.float32)
        ones16 = jnp.ones((16,), jnp.float32)

        def initz(i, _):
            zb_v[i, :] = zeros16
            return 0

        lax.fori_loop(0, RPS, initz, 0)

        def inito(i, _):
            ones_v[i, :] = ones16
            return 0

        lax.fori_loop(0, BE, inito, 0)
        pltpu.sync_copy(zb_v, acc_s.at[pl.ds(sid * RPS, RPS)])
        plsc.subcore_barrier()

        def chunk(j, _):
            base = wid * ept + j * BE
            pltpu.sync_copy(col_hbm.at[pl.ds(base, BE)], col_v)
            pltpu.sync_copy(ones_v, acc_s.at[col_v], add=True)
            return 0

        lax.fori_loop(0, nch, chunk, 0)
        plsc.subcore_barrier()
        pltpu.sync_copy(acc_s.at[pl.ds(sid * RPS, RPS)], zb_v)
        pltpu.sync_copy(zb_v, deg_hbm.at[cid, pl.ds(sid * RPS, RPS)])

    return k(col_p)


def _conv(L, row_p, col_p, Q, Kc, Vc):
    ept = row_p.shape[0] // NW
    nch = ept // BE
    CK = L * CH
    mesh = plsc.VectorSubcoreMesh(core_axis_name="c", subcore_axis_name="s")

    @functools.partial(
        pl.kernel,
        out_type=jax.ShapeDtypeStruct((NCORE, NPAD, CH), jnp.float32),
        mesh=mesh,
        compiler_params=pltpu.CompilerParams(needs_layout_passes=False,
                                             use_tc_tiling_on_sc=False),
        scratch_types=[
            pltpu.VMEM((BE,), jnp.int32),
            pltpu.VMEM((BE,), jnp.int32),
            pltpu.VMEM((BE, CH), jnp.float32),
            pltpu.VMEM((BE, CK), jnp.float32),
            pltpu.VMEM((BE, CK), jnp.float32),
            pltpu.MemorySpace.VMEM_SHARED((NPAD, CH), jnp.float32),
            pltpu.SemaphoreType.DMA,
        ],
    )
    def k(row_hbm, col_hbm, q_hbm, k_hbm, v_hbm, out_hbm,
          row_v, col_v, q_v, k_v, v_v, acc_s, sem):
        cid = lax.axis_index("c")
        sid = lax.axis_index("s")
        wid = sid * NCORE + cid
        zeros16 = jnp.zeros((16,), jnp.float32)

        def initz(i, _):
            q_v[i // HEADS, pl.ds((i % HEADS) * DH, DH)] = zeros16
            return 0

        lax.fori_loop(0, BE * HEADS, initz, 0)
        for t in range(RPS // BE):
            pltpu.sync_copy(q_v, acc_s.at[pl.ds(sid * RPS + t * BE, BE)])
        plsc.subcore_barrier()

        def chunk(j, _):
            base = wid * ept + j * BE
            pltpu.sync_copy(row_hbm.at[pl.ds(base, BE)], row_v)
            pltpu.sync_copy(col_hbm.at[pl.ds(base, BE)], col_v)
            cp_q = pltpu.async_copy(q_hbm.at[col_v], q_v, sem)
            cp_k = pltpu.async_copy(k_hbm.at[row_v], k_v, sem)
            cp_v = pltpu.async_copy(v_hbm.at[row_v], v_v, sem)
            cp_q.wait()
            cp_k.wait()
            cp_v.wait()

            def edge(e, _):
                zv = jnp.zeros((DH,), jnp.float32)
                for h in range(HEADS):
                    o = h * DH
                    q = q_v[e, pl.ds(o, DH)]
                    if L == 1:
                        k0 = k_v[e, pl.ds(o, DH)]
                        s0 = _dot16(q, k0) * 0.25
                        m = jnp.maximum(s0, zv)
                        e0 = jnp.exp(s0 - m)
                        em = jnp.exp(-m)
                        mh = (e0 / (e0 + em)) * v_v[e, pl.ds(o, DH)]
                    else:
                        k0 = k_v[e, pl.ds(o, DH)]
                        k1 = k_v[e, pl.ds(CH + o, DH)]
                        s0 = _dot16(q, k0) * 0.25
                        s1 = _dot16(q, k1) * 0.25
                        m = jnp.maximum(jnp.maximum(s0, s1), zv)
                        e0 = jnp.exp(s0 - m)
                        e1 = jnp.exp(s1 - m)
                        em = jnp.exp(-m)
                        v0 = v_v[e, pl.ds(o, DH)]
                        v1 = v_v[e, pl.ds(CH + o, DH)]
                        mh = (e0 * v0 + e1 * v1) / (e0 + e1 + em)
                    q_v[e, pl.ds(o, DH)] = mh
                return 0

            lax.fori_loop(0, BE, edge, 0)
            pltpu.sync_copy(q_v, acc_s.at[col_v], add=True)
            return 0

        lax.fori_loop(0, nch, chunk, 0)
        plsc.subcore_barrier()
        for t in range(RPS // BE):
            r0 = sid * RPS + t * BE
            pltpu.sync_copy(acc_s.at[pl.ds(r0, BE)], q_v)
            pltpu.sync_copy(q_v, out_hbm.at[cid, pl.ds(r0, BE)])

    return k(row_p, col_p, Q, Kc, Vc)


def _tc1(x_p, W1, b1, Wq, bq, Wk, bk, Wv, bv, deg2):
    grid = (NPAD // BN,)
    nreal_ref = None

    def body(x_ref, w1_ref, b1_ref, wq_ref, bq_ref, wk_ref, bk_ref,
             wv_ref, bv_ref, deg_ref, h_ref, q_ref, k_ref, v_ref, dis_ref):
        i = pl.program_id(0)
        h = jnp.maximum(x_ref[...] @ w1_ref[...] + b1_ref[...], 0.0)
        deg = deg_ref[0, :, 0:1] + deg_ref[1, :, 0:1] + 1.0
        rows = i * BN + lax.broadcasted_iota(jnp.int32, (BN, 1), 0)
        dis = jnp.where(rows < _NREAL, lax.rsqrt(deg), 0.0)
        h_ref[...] = h
        q_ref[...] = h @ wq_ref[...] + bq_ref[...]
        k_ref[...] = h @ wk_ref[...] + bk_ref[...]
        v_ref[...] = (h @ wv_ref[...] + bv_ref[...]) * dis
        dis_ref[...] = jnp.broadcast_to(dis, (BN, CH))

    full = lambda s: pl.BlockSpec(s, lambda i: (0,) * len(s))
    rowb = pl.BlockSpec((BN, CH), lambda i: (i, 0))
    return pl.pallas_call(
        body,
        grid=grid,
        in_specs=[rowb, full((CH, CH)), full((1, CH)), full((CH, CH)),
                  full((1, CH)), full((CH, CH)), full((1, CH)),
                  full((CH, CH)), full((1, CH)),
                  pl.BlockSpec((2, BN, 16), lambda i: (0, i, 0))],
        out_specs=[rowb, rowb, rowb, rowb, rowb],
        out_shape=[jax.ShapeDtypeStruct((NPAD, CH), jnp.float32)] * 5,
    )(x_p, W1, b1, Wq, bq, Wk, bk, Wv, bv, deg2)


def _tc2(h, agg0, dis, Wq, bq, Wk, bk, Wv, bv):
    grid = (NPAD // BN,)

    def body(h_ref, a_ref, dis_ref, wq_ref, bq_ref, wk_ref, bk_ref,
             wv_ref, bv_ref, q_ref, k_ref, v_ref):
        dis = dis_ref[...]
        h = h_ref[...]
        h1 = jnp.maximum(dis * (a_ref[0] + a_ref[1]), 0.0)
        q_ref[...] = h1 @ wq_ref[...] + bq_ref[...]
        k_ref[:, 0:CH] = h @ wk_ref[...] + bk_ref[...]
        k_ref[:, CH:2 * CH] = h1 @ wk_ref[...] + bk_ref[...]
        v_ref[:, 0:CH] = (h @ wv_ref[...] + bv_ref[...]) * dis
        v_ref[:, CH:2 * CH] = (h1 @ wv_ref[...] + bv_ref[...]) * dis

    full = lambda s: pl.BlockSpec(s, lambda i: (0,) * len(s))
    rowb = pl.BlockSpec((BN, CH), lambda i: (i, 0))
    rowb2 = pl.BlockSpec((BN, 2 * CH), lambda i: (i, 0))
    return pl.pallas_call(
        body,
        grid=grid,
        in_specs=[rowb, pl.BlockSpec((2, BN, CH), lambda i: (0, i, 0)), rowb,
                  full((CH, CH)), full((1, CH)), full((CH, CH)), full((1, CH)),
                  full((CH, CH)), full((1, CH))],
        out_specs=[rowb, rowb2, rowb2],
        out_shape=[jax.ShapeDtypeStruct((NPAD, CH), jnp.float32),
                   jax.ShapeDtypeStruct((NPAD, 2 * CH), jnp.float32),
                   jax.ShapeDtypeStruct((NPAD, 2 * CH), jnp.float32)],
    )(h, agg0, dis, Wq, bq, Wk, bk, Wv, bv)


def _tc3(agg1, dis, W2p, b2p, nclass):
    grid = (NPAD // BN,)

    def body(a_ref, dis_ref, w2_ref, b2_ref, o_ref):
        h2 = jnp.maximum(dis_ref[...] * (a_ref[0] + a_ref[1]), 0.0)
        lg = h2 @ w2_ref[...] + b2_ref[...]
        colm = lax.broadcasted_iota(jnp.int32, (BN, CH), 1) < nclass
        lgm = jnp.where(colm, lg, -1e30)
        mx = jnp.max(lgm, axis=1, keepdims=True)
        sm = jnp.sum(jnp.exp(lgm - mx), axis=1, keepdims=True)
        o_ref[...] = lg - mx - jnp.log(sm)

    full = lambda s: pl.BlockSpec(s, lambda i: (0,) * len(s))
    rowb = pl.BlockSpec((BN, CH), lambda i: (i, 0))
    return pl.pallas_call(
        body,
        grid=grid,
        in_specs=[pl.BlockSpec((2, BN, CH), lambda i: (0, i, 0)), rowb,
                  full((CH, CH)), full((1, CH))],
        out_specs=rowb,
        out_shape=jax.ShapeDtypeStruct((NPAD, CH), jnp.float32),
    )(agg1, dis, W2p, b2p)


_NREAL = 10000


def kernel(x, edge_index, W1, b1, Wq0, bq0, Wk0, bk0, Wv0, bv0,
           Wq1, bq1, Wk1, bk1, Wv1, bv1, W2, b2):
    global _NREAL
    N, F = x.shape
    _NREAL = N
    E = edge_index.shape[1]
    nclass = W2.shape[1]
    idt = edge_index.dtype

    loops = jnp.arange(N, dtype=idt)
    row = jnp.concatenate([edge_index[0], loops])
    col = jnp.concatenate([edge_index[1], loops])
    etot = E + N
    epad = ((etot + NW * BE - 1) // (NW * BE)) * (NW * BE)
    padidx = jnp.full((epad - etot,), NPAD - 1, idt)
    row_p = jnp.concatenate([row, padidx]).astype(jnp.int32)
    col_p = jnp.concatenate([col, padidx]).astype(jnp.int32)

    x_p = jnp.pad(x, ((0, NPAD - N), (0, 0)))
    r2 = lambda b: b.reshape(1, CH)
    bd = _block_diag

    deg2 = _degree(col_p)
    h, q0, k0, v0p, dis = _tc1(x_p, W1, r2(b1), bd(Wq0), r2(bq0),
                               bd(Wk0), r2(bk0), bd(Wv0), r2(bv0), deg2)
    agg0 = _conv(1, row_p, col_p, q0, k0, v0p)
    q1, k1c, v1c = _tc2(h, agg0, dis, bd(Wq1), r2(bq1), bd(Wk1), r2(bk1),
                        bd(Wv1), r2(bv1))
    agg1 = _conv(2, row_p, col_p, q1, k1c, v1c)
    W2p = jnp.pad(W2, ((0, 0), (0, CH - nclass)))
    b2p = jnp.pad(b2, (0, CH - nclass)).reshape(1, CH)
    out = _tc3(agg1, dis, W2p, b2p, nclass)
    return out[:N, :nclass]


# speedup vs baseline: 2.4794x; 1.0075x over previous
import functools

import jax
import jax.numpy as jnp
from jax import lax
from jax.experimental import pallas as pl
from jax.experimental.pallas import tpu as pltpu
from jax.experimental.pallas import tpu_sc as plsc

NCORE = 2
NSUB = 16
NW = NCORE * NSUB
CH = 128
HEADS = 8
DH = 16
NPAD = 10240
BN = 512
BE = 64
RPS = NPAD // NSUB

_GDN = lax.GatherDimensionNumbers(
    offset_dims=(), collapsed_slice_dims=(0,), start_index_map=(0,))


def _dot16(a, b):
    p = a * b
    for k in (8, 4, 2, 1):
        perm = (jnp.arange(16, dtype=jnp.int32) ^ k)[:, None]
        p = p + lax.gather(p, perm, _GDN, (1,),
                           mode=lax.GatherScatterMode.PROMISE_IN_BOUNDS)
    return p


def _block_diag(W):
    G, ci, co = W.shape
    out = jnp.zeros((G * ci, G * co), W.dtype)
    for g in range(G):
        out = out.at[g * ci:(g + 1) * ci, g * co:(g + 1) * co].set(W[g])
    return out


def _degree(col_p):
    ept = col_p.shape[0] // NW
    nch = ept // BE
    mesh = plsc.VectorSubcoreMesh(core_axis_name="c", subcore_axis_name="s")

    @functools.partial(
        pl.kernel,
        out_type=jax.ShapeDtypeStruct((NCORE, NPAD, 16), jnp.float32),
        mesh=mesh,
        compiler_params=pltpu.CompilerParams(needs_layout_passes=False,
                                             use_tc_tiling_on_sc=False),
        scratch_types=[
            pltpu.VMEM((BE,), jnp.int32),
            pltpu.VMEM((BE, 16), jnp.float32),
            pltpu.VMEM((RPS, 16), jnp.float32),
            pltpu.MemorySpace.VMEM_SHARED((NPAD, 16), jnp.float32),
        ],
    )
    def k(col_hbm, deg_hbm, col_v, ones_v, zb_v, acc_s):
        cid = lax.axis_index("c")
        sid = lax.axis_index("s")
        wid = sid * NCORE + cid
        zeros16 = jnp.zeros((16,), jnp.float32)
        ones16 = jnp.ones((16,), jnp.float32)

        def initz(i, _):
            zb_v[i, :] = zeros16
            return 0

        lax.fori_loop(0, RPS, initz, 0)

        def inito(i, _):
            ones_v[i, :] = ones16
            return 0

        lax.fori_loop(0, BE, inito, 0)
        pltpu.sync_copy(zb_v, acc_s.at[pl.ds(sid * RPS, RPS)])
        plsc.subcore_barrier()

        def chunk(j, _):
            base = wid * ept + j * BE
            pltpu.sync_copy(col_hbm.at[pl.ds(base, BE)], col_v)
            pltpu.sync_copy(ones_v, acc_s.at[col_v], add=True)
            return 0

        lax.fori_loop(0, nch, chunk, 0)
        plsc.subcore_barrier()
        pltpu.sync_copy(acc_s.at[pl.ds(sid * RPS, RPS)], zb_v)
        pltpu.sync_copy(zb_v, deg_hbm.at[cid, pl.ds(sid * RPS, RPS)])

    return k(col_p)


def _conv(L, row_p, col_p, Q, Kc, Vc):
    ept = row_p.shape[0] // NW
    nch = ept // BE
    CK = L * CH
    mesh = plsc.VectorSubcoreMesh(core_axis_name="c", subcore_axis_name="s")

    @functools.partial(
        pl.kernel,
        out_type=jax.ShapeDtypeStruct((NCORE, NPAD, CH), jnp.float32),
        mesh=mesh,
        compiler_params=pltpu.CompilerParams(needs_layout_passes=False,
                                             use_tc_tiling_on_sc=False),
        scratch_types=[
            pltpu.VMEM((BE,), jnp.int32),
            pltpu.VMEM((BE,), jnp.int32),
            pltpu.VMEM((BE, CH), jnp.float32),
            pltpu.VMEM((BE, CK), jnp.float32),
            pltpu.VMEM((BE, CK), jnp.float32),
            pltpu.MemorySpace.VMEM_SHARED((NPAD, CH), jnp.float32),
            pltpu.SemaphoreType.DMA,
        ],
    )
    def k(row_hbm, col_hbm, q_hbm, k_hbm, v_hbm, out_hbm,
          row_v, col_v, q_v, k_v, v_v, acc_s, sem):
        cid = lax.axis_index("c")
        sid = lax.axis_index("s")
        wid = sid * NCORE + cid
        zeros16 = jnp.zeros((16,), jnp.float32)

        def initz(i, _):
            q_v[i // HEADS, pl.ds((i % HEADS) * DH, DH)] = zeros16
            return 0

        lax.fori_loop(0, BE * HEADS, initz, 0)
        for t in range(RPS // BE):
            pltpu.sync_copy(q_v, acc_s.at[pl.ds(sid * RPS + t * BE, BE)])
        plsc.subcore_barrier()

        def chunk(j, _):
            base = wid * ept + j * BE
            pltpu.sync_copy(row_hbm.at[pl.ds(base, BE)], row_v)
            pltpu.sync_copy(col_hbm.at[pl.ds(base, BE)], col_v)
            cp_q = pltpu.async_copy(q_hbm.at[col_v], q_v, sem)
            cp_k = pltpu.async_copy(k_hbm.at[row_v], k_v, sem)
            cp_v = pltpu.async_copy(v_hbm.at[row_v], v_v, sem)
            cp_q.wait()
            cp_k.wait()
            cp_v.wait()

            def edge2(e2, _):
                zv = jnp.zeros((DH,), jnp.float32)
                for u in range(2):
                    e = e2 * 2 + u
                    _one_edge(e, zv)
                return 0

            def _one_edge(e, zv):
                for h in range(HEADS):
                    o = h * DH
                    q = q_v[e, pl.ds(o, DH)]
                    if L == 1:
                        k0 = k_v[e, pl.ds(o, DH)]
                        s0 = _dot16(q, k0) * 0.25
                        m = jnp.maximum(s0, zv)
                        e0 = jnp.exp(s0 - m)
                        em = jnp.exp(-m)
                        mh = (e0 / (e0 + em)) * v_v[e, pl.ds(o, DH)]
                    else:
                        k0 = k_v[e, pl.ds(o, DH)]
                        k1 = k_v[e, pl.ds(CH + o, DH)]
                        s0 = _dot16(q, k0) * 0.25
                        s1 = _dot16(q, k1) * 0.25
                        m = jnp.maximum(jnp.maximum(s0, s1), zv)
                        e0 = jnp.exp(s0 - m)
                        e1 = jnp.exp(s1 - m)
                        em = jnp.exp(-m)
                        v0 = v_v[e, pl.ds(o, DH)]
                        v1 = v_v[e, pl.ds(CH + o, DH)]
                        mh = (e0 * v0 + e1 * v1) / (e0 + e1 + em)
                    q_v[e, pl.ds(o, DH)] = mh

            lax.fori_loop(0, BE // 2, edge2, 0)
            pltpu.sync_copy(q_v, acc_s.at[col_v], add=True)
            return 0

        lax.fori_loop(0, nch, chunk, 0)
        plsc.subcore_barrier()
        for t in range(RPS // BE):
            r0 = sid * RPS + t * BE
            pltpu.sync_copy(acc_s.at[pl.ds(r0, BE)], q_v)
            pltpu.sync_copy(q_v, out_hbm.at[cid, pl.ds(r0, BE)])

    return k(row_p, col_p, Q, Kc, Vc)


def _tc1(x_p, W1, b1, Wq, bq, Wk, bk, Wv, bv, deg2):
    grid = (NPAD // BN,)
    nreal_ref = None

    def body(x_ref, w1_ref, b1_ref, wq_ref, bq_ref, wk_ref, bk_ref,
             wv_ref, bv_ref, deg_ref, h_ref, q_ref, k_ref, v_ref, dis_ref):
        i = pl.program_id(0)
        h = jnp.maximum(x_ref[...] @ w1_ref[...] + b1_ref[...], 0.0)
        deg = deg_ref[0, :, 0:1] + deg_ref[1, :, 0:1] + 1.0
        rows = i * BN + lax.broadcasted_iota(jnp.int32, (BN, 1), 0)
        dis = jnp.where(rows < _NREAL, lax.rsqrt(deg), 0.0)
        h_ref[...] = h
        q_ref[...] = h @ wq_ref[...] + bq_ref[...]
        k_ref[...] = h @ wk_ref[...] + bk_ref[...]
        v_ref[...] = (h @ wv_ref[...] + bv_ref[...]) * dis
        dis_ref[...] = jnp.broadcast_to(dis, (BN, CH))

    full = lambda s: pl.BlockSpec(s, lambda i: (0,) * len(s))
    rowb = pl.BlockSpec((BN, CH), lambda i: (i, 0))
    return pl.pallas_call(
        body,
        grid=grid,
        in_specs=[rowb, full((CH, CH)), full((1, CH)), full((CH, CH)),
                  full((1, CH)), full((CH, CH)), full((1, CH)),
                  full((CH, CH)), full((1, CH)),
                  pl.BlockSpec((2, BN, 16), lambda i: (0, i, 0))],
        out_specs=[rowb, rowb, rowb, rowb, rowb],
        out_shape=[jax.ShapeDtypeStruct((NPAD, CH), jnp.float32)] * 5,
    )(x_p, W1, b1, Wq, bq, Wk, bk, Wv, bv, deg2)


def _tc2(h, agg0, dis, Wq, bq, Wk, bk, Wv, bv):
    grid = (NPAD // BN,)

    def body(h_ref, a_ref, dis_ref, wq_ref, bq_ref, wk_ref, bk_ref,
             wv_ref, bv_ref, q_ref, k_ref, v_ref):
        dis = dis_ref[...]
        h = h_ref[...]
        h1 = jnp.maximum(dis * (a_ref[0] + a_ref[1]), 0.0)
        q_ref[...] = h1 @ wq_ref[...] + bq_ref[...]
        k_ref[:, 0:CH] = h @ wk_ref[...] + bk_ref[...]
        k_ref[:, CH:2 * CH] = h1 @ wk_ref[...] + bk_ref[...]
        v_ref[:, 0:CH] = (h @ wv_ref[...] + bv_ref[...]) * dis
        v_ref[:, CH:2 * CH] = (h1 @ wv_ref[...] + bv_ref[...]) * dis

    full = lambda s: pl.BlockSpec(s, lambda i: (0,) * len(s))
    rowb = pl.BlockSpec((BN, CH), lambda i: (i, 0))
    rowb2 = pl.BlockSpec((BN, 2 * CH), lambda i: (i, 0))
    return pl.pallas_call(
        body,
        grid=grid,
        in_specs=[rowb, pl.BlockSpec((2, BN, CH), lambda i: (0, i, 0)), rowb,
                  full((CH, CH)), full((1, CH)), full((CH, CH)), full((1, CH)),
                  full((CH, CH)), full((1, CH))],
        out_specs=[rowb, rowb2, rowb2],
        out_shape=[jax.ShapeDtypeStruct((NPAD, CH), jnp.float32),
                   jax.ShapeDtypeStruct((NPAD, 2 * CH), jnp.float32),
                   jax.ShapeDtypeStruct((NPAD, 2 * CH), jnp.float32)],
    )(h, agg0, dis, Wq, bq, Wk, bk, Wv, bv)


def _tc3(agg1, dis, W2p, b2p, nclass):
    grid = (NPAD // BN,)

    def body(a_ref, dis_ref, w2_ref, b2_ref, o_ref):
        h2 = jnp.maximum(dis_ref[...] * (a_ref[0] + a_ref[1]), 0.0)
        lg = h2 @ w2_ref[...] + b2_ref[...]
        colm = lax.broadcasted_iota(jnp.int32, (BN, CH), 1) < nclass
        lgm = jnp.where(colm, lg, -1e30)
        mx = jnp.max(lgm, axis=1, keepdims=True)
        sm = jnp.sum(jnp.exp(lgm - mx), axis=1, keepdims=True)
        o_ref[...] = lg - mx - jnp.log(sm)

    full = lambda s: pl.BlockSpec(s, lambda i: (0,) * len(s))
    rowb = pl.BlockSpec((BN, CH), lambda i: (i, 0))
    return pl.pallas_call(
        body,
        grid=grid,
        in_specs=[pl.BlockSpec((2, BN, CH), lambda i: (0, i, 0)), rowb,
                  full((CH, CH)), full((1, CH))],
        out_specs=rowb,
        out_shape=jax.ShapeDtypeStruct((NPAD, CH), jnp.float32),
    )(agg1, dis, W2p, b2p)


_NREAL = 10000


def kernel(x, edge_index, W1, b1, Wq0, bq0, Wk0, bk0, Wv0, bv0,
           Wq1, bq1, Wk1, bk1, Wv1, bv1, W2, b2):
    global _NREAL
    N, F = x.shape
    _NREAL = N
    E = edge_index.shape[1]
    nclass = W2.shape[1]
    idt = edge_index.dtype

    loops = jnp.arange(N, dtype=idt)
    row = jnp.concatenate([edge_index[0], loops])
    col = jnp.concatenate([edge_index[1], loops])
    etot = E + N
    epad = ((etot + NW * BE - 1) // (NW * BE)) * (NW * BE)
    padidx = jnp.full((epad - etot,), NPAD - 1, idt)
    row_p = jnp.concatenate([row, padidx]).astype(jnp.int32)
    col_p = jnp.concatenate([col, padidx]).astype(jnp.int32)

    x_p = jnp.pad(x, ((0, NPAD - N), (0, 0)))
    r2 = lambda b: b.reshape(1, CH)
    bd = _block_diag

    deg2 = _degree(col_p)
    h, q0, k0, v0p, dis = _tc1(x_p, W1, r2(b1), bd(Wq0), r2(bq0),
                               bd(Wk0), r2(bk0), bd(Wv0), r2(bv0), deg2)
    agg0 = _conv(1, row_p, col_p, q0, k0, v0p)
    q1, k1c, v1c = _tc2(h, agg0, dis, bd(Wq1), r2(bq1), bd(Wk1), r2(bk1),
                        bd(Wv1), r2(bv1))
    agg1 = _conv(2, row_p, col_p, q1, k1c, v1c)
    W2p = jnp.pad(W2, ((0, 0), (0, CH - nclass)))
    b2p = jnp.pad(b2, (0, CH - nclass)).reshape(1, CH)
    out = _tc3(agg1, dis, W2p, b2p, nclass)
    return out[:N, :nclass]


# speedup vs baseline: 2.8022x; 1.1302x over previous
import functools

import jax
import jax.numpy as jnp
from jax import lax
from jax.experimental import pallas as pl
from jax.experimental.pallas import tpu as pltpu
from jax.experimental.pallas import tpu_sc as plsc

NCORE = 2
NSUB = 16
NW = NCORE * NSUB
CH = 128
HEADS = 8
DH = 16
NPAD = 10240
BN = 512
BE = 32
RPS = NPAD // NSUB

_GDN = lax.GatherDimensionNumbers(
    offset_dims=(), collapsed_slice_dims=(0,), start_index_map=(0,))


def _dot16(a, b):
    p = a * b
    for k in (8, 4, 2, 1):
        perm = (jnp.arange(16, dtype=jnp.int32) ^ k)[:, None]
        p = p + lax.gather(p, perm, _GDN, (1,),
                           mode=lax.GatherScatterMode.PROMISE_IN_BOUNDS)
    return p


def _block_diag(W):
    G, ci, co = W.shape
    out = jnp.zeros((G * ci, G * co), W.dtype)
    for g in range(G):
        out = out.at[g * ci:(g + 1) * ci, g * co:(g + 1) * co].set(W[g])
    return out


def _degree(col_p):
    ept = col_p.shape[0] // NW
    nch = ept // BE
    mesh = plsc.VectorSubcoreMesh(core_axis_name="c", subcore_axis_name="s")

    @functools.partial(
        pl.kernel,
        out_type=jax.ShapeDtypeStruct((NCORE, NPAD, 16), jnp.float32),
        mesh=mesh,
        compiler_params=pltpu.CompilerParams(needs_layout_passes=False,
                                             use_tc_tiling_on_sc=False),
        scratch_types=[
            pltpu.VMEM((BE,), jnp.int32),
            pltpu.VMEM((BE, 16), jnp.float32),
            pltpu.VMEM((RPS, 16), jnp.float32),
            pltpu.MemorySpace.VMEM_SHARED((NPAD, 16), jnp.float32),
        ],
    )
    def k(col_hbm, deg_hbm, col_v, ones_v, zb_v, acc_s):
        cid = lax.axis_index("c")
        sid = lax.axis_index("s")
        wid = sid * NCORE + cid
        zeros16 = jnp.zeros((16,), jnp.float32)
        ones16 = jnp.ones((16,), jnp.float32)

        def initz(i, _):
            zb_v[i, :] = zeros16
            return 0

        lax.fori_loop(0, RPS, initz, 0)

        def inito(i, _):
            ones_v[i, :] = ones16
            return 0

        lax.fori_loop(0, BE, inito, 0)
        pltpu.sync_copy(zb_v, acc_s.at[pl.ds(sid * RPS, RPS)])
        plsc.subcore_barrier()

        def chunk(j, _):
            base = wid * ept + j * BE
            pltpu.sync_copy(col_hbm.at[pl.ds(base, BE)], col_v)
            pltpu.sync_copy(ones_v, acc_s.at[col_v], add=True)
            return 0

        lax.fori_loop(0, nch, chunk, 0)
        plsc.subcore_barrier()
        pltpu.sync_copy(acc_s.at[pl.ds(sid * RPS, RPS)], zb_v)
        pltpu.sync_copy(zb_v, deg_hbm.at[cid, pl.ds(sid * RPS, RPS)])

    return k(col_p)


def _conv(L, row_p, col_p, Q, Kc, Vc):
    ept = row_p.shape[0] // NW
    nch = ept // BE
    CK = L * CH
    mesh = plsc.VectorSubcoreMesh(core_axis_name="c", subcore_axis_name="s")

    @functools.partial(
        pl.kernel,
        out_type=jax.ShapeDtypeStruct((NCORE, NPAD, CH), jnp.float32),
        mesh=mesh,
        compiler_params=pltpu.CompilerParams(needs_layout_passes=False,
                                             use_tc_tiling_on_sc=False),
        scratch_types=[
            pltpu.VMEM((2, BE), jnp.int32),
            pltpu.VMEM((2, BE), jnp.int32),
            pltpu.VMEM((2, BE, CH), jnp.float32),
            pltpu.VMEM((2, BE, CK), jnp.float32),
            pltpu.VMEM((2, BE, CK), jnp.float32),
            pltpu.MemorySpace.VMEM_SHARED((NPAD, CH), jnp.float32),
            pltpu.SemaphoreType.DMA,
            pltpu.SemaphoreType.DMA,
        ],
    )
    def k(row_hbm, col_hbm, q_hbm, k_hbm, v_hbm, out_hbm,
          row_b, col_b, q_b, k_b, v_b, acc_s, sem0, sem1):
        cid = lax.axis_index("c")
        sid = lax.axis_index("s")
        wid = sid * NCORE + cid
        zeros16 = jnp.zeros((16,), jnp.float32)
        sems = (sem0, sem1)

        def initz(i, _):
            q_b[0, i // HEADS, pl.ds((i % HEADS) * DH, DH)] = zeros16
            return 0

        lax.fori_loop(0, BE * HEADS, initz, 0)
        for t in range(RPS // BE):
            pltpu.sync_copy(q_b.at[0], acc_s.at[pl.ds(sid * RPS + t * BE, BE)])
        plsc.subcore_barrier()

        def fetch(j, b):
            base = wid * ept + j * BE
            pltpu.sync_copy(row_hbm.at[pl.ds(base, BE)], row_b.at[b])
            pltpu.sync_copy(col_hbm.at[pl.ds(base, BE)], col_b.at[b])
            pltpu.async_copy(q_hbm.at[col_b.at[b]], q_b.at[b], sems[b])
            pltpu.async_copy(k_hbm.at[row_b.at[b]], k_b.at[b], sems[b])
            pltpu.async_copy(v_hbm.at[row_b.at[b]], v_b.at[b], sems[b])

        def compute(b):
            q_v, k_v, v_v = q_b.at[b], k_b.at[b], v_b.at[b]

            def _one_edge(e, zv):
                for h in range(HEADS):
                    o = h * DH
                    q = q_v[e, pl.ds(o, DH)]
                    if L == 1:
                        k0 = k_v[e, pl.ds(o, DH)]
                        s0 = _dot16(q, k0) * 0.25
                        m = jnp.maximum(s0, zv)
                        e0 = jnp.exp(s0 - m)
                        em = jnp.exp(-m)
                        mh = (e0 / (e0 + em)) * v_v[e, pl.ds(o, DH)]
                    else:
                        k0 = k_v[e, pl.ds(o, DH)]
                        k1 = k_v[e, pl.ds(CH + o, DH)]
                        s0 = _dot16(q, k0) * 0.25
                        s1 = _dot16(q, k1) * 0.25
                        m = jnp.maximum(jnp.maximum(s0, s1), zv)
                        e0 = jnp.exp(s0 - m)
                        e1 = jnp.exp(s1 - m)
                        em = jnp.exp(-m)
                        v0 = v_v[e, pl.ds(o, DH)]
                        v1 = v_v[e, pl.ds(CH + o, DH)]
                        mh = (e0 * v0 + e1 * v1) / (e0 + e1 + em)
                    q_v[e, pl.ds(o, DH)] = mh

            def edge2(e2, _):
                zv = jnp.zeros((DH,), jnp.float32)
                for u in range(2):
                    _one_edge(e2 * 2 + u, zv)
                return 0

            lax.fori_loop(0, BE // 2, edge2, 0)

        fetch(0, 0)

        def pair(j2, _):
            for b in range(2):
                j = j2 * 2 + b
                nb = 1 - b

                @pl.when(j + 1 < nch)
                def _():
                    fetch(j + 1, nb)

                pltpu.make_async_copy(q_hbm.at[col_b.at[b]], q_b.at[b],
                                      sems[b]).wait()
                pltpu.make_async_copy(k_hbm.at[row_b.at[b]], k_b.at[b],
                                      sems[b]).wait()
                pltpu.make_async_copy(v_hbm.at[row_b.at[b]], v_b.at[b],
                                      sems[b]).wait()
                compute(b)
                pltpu.sync_copy(q_b.at[b], acc_s.at[col_b.at[b]], add=True)
            return 0

        lax.fori_loop(0, nch // 2, pair, 0)
        plsc.subcore_barrier()
        for t in range(RPS // BE):
            r0 = sid * RPS + t * BE
            pltpu.sync_copy(acc_s.at[pl.ds(r0, BE)], q_b.at[0])
            pltpu.sync_copy(q_b.at[0], out_hbm.at[cid, pl.ds(r0, BE)])

    return k(row_p, col_p, Q, Kc, Vc)


def _tc1(x_p, W1, b1, Wq, bq, Wk, bk, Wv, bv, deg2):
    grid = (NPAD // BN,)
    nreal_ref = None

    def body(x_ref, w1_ref, b1_ref, wq_ref, bq_ref, wk_ref, bk_ref,
             wv_ref, bv_ref, deg_ref, h_ref, q_ref, k_ref, v_ref, dis_ref):
        i = pl.program_id(0)
        h = jnp.maximum(x_ref[...] @ w1_ref[...] + b1_ref[...], 0.0)
        deg = deg_ref[0, :, 0:1] + deg_ref[1, :, 0:1] + 1.0
        rows = i * BN + lax.broadcasted_iota(jnp.int32, (BN, 1), 0)
        dis = jnp.where(rows < _NREAL, lax.rsqrt(deg), 0.0)
        h_ref[...] = h
        q_ref[...] = h @ wq_ref[...] + bq_ref[...]
        k_ref[...] = h @ wk_ref[...] + bk_ref[...]
        v_ref[...] = (h @ wv_ref[...] + bv_ref[...]) * dis
        dis_ref[...] = jnp.broadcast_to(dis, (BN, CH))

    full = lambda s: pl.BlockSpec(s, lambda i: (0,) * len(s))
    rowb = pl.BlockSpec((BN, CH), lambda i: (i, 0))
    return pl.pallas_call(
        body,
        grid=grid,
        in_specs=[rowb, full((CH, CH)), full((1, CH)), full((CH, CH)),
                  full((1, CH)), full((CH, CH)), full((1, CH)),
                  full((CH, CH)), full((1, CH)),
                  pl.BlockSpec((2, BN, 16), lambda i: (0, i, 0))],
        out_specs=[rowb, rowb, rowb, rowb, rowb],
        out_shape=[jax.ShapeDtypeStruct((NPAD, CH), jnp.float32)] * 5,
    )(x_p, W1, b1, Wq, bq, Wk, bk, Wv, bv, deg2)


def _tc2(h, agg0, dis, Wq, bq, Wk, bk, Wv, bv):
    grid = (NPAD // BN,)

    def body(h_ref, a_ref, dis_ref, wq_ref, bq_ref, wk_ref, bk_ref,
             wv_ref, bv_ref, q_ref, k_ref, v_ref):
        dis = dis_ref[...]
        h = h_ref[...]
        h1 = jnp.maximum(dis * (a_ref[0] + a_ref[1]), 0.0)
        q_ref[...] = h1 @ wq_ref[...] + bq_ref[...]
        k_ref[:, 0:CH] = h @ wk_ref[...] + bk_ref[...]
        k_ref[:, CH:2 * CH] = h1 @ wk_ref[...] + bk_ref[...]
        v_ref[:, 0:CH] = (h @ wv_ref[...] + bv_ref[...]) * dis
        v_ref[:, CH:2 * CH] = (h1 @ wv_ref[...] + bv_ref[...]) * dis

    full = lambda s: pl.BlockSpec(s, lambda i: (0,) * len(s))
    rowb = pl.BlockSpec((BN, CH), lambda i: (i, 0))
    rowb2 = pl.BlockSpec((BN, 2 * CH), lambda i: (i, 0))
    return pl.pallas_call(
        body,
        grid=grid,
        in_specs=[rowb, pl.BlockSpec((2, BN, CH), lambda i: (0, i, 0)), rowb,
                  full((CH, CH)), full((1, CH)), full((CH, CH)), full((1, CH)),
                  full((CH, CH)), full((1, CH))],
        out_specs=[rowb, rowb2, rowb2],
        out_shape=[jax.ShapeDtypeStruct((NPAD, CH), jnp.float32),
                   jax.ShapeDtypeStruct((NPAD, 2 * CH), jnp.float32),
                   jax.ShapeDtypeStruct((NPAD, 2 * CH), jnp.float32)],
    )(h, agg0, dis, Wq, bq, Wk, bk, Wv, bv)


def _tc3(agg1, dis, W2p, b2p, nclass):
    grid = (NPAD // BN,)

    def body(a_ref, dis_ref, w2_ref, b2_ref, o_ref):
        h2 = jnp.maximum(dis_ref[...] * (a_ref[0] + a_ref[1]), 0.0)
        lg = h2 @ w2_ref[...] + b2_ref[...]
        colm = lax.broadcasted_iota(jnp.int32, (BN, CH), 1) < nclass
        lgm = jnp.where(colm, lg, -1e30)
        mx = jnp.max(lgm, axis=1, keepdims=True)
        sm = jnp.sum(jnp.exp(lgm - mx), axis=1, keepdims=True)
        o_ref[...] = lg - mx - jnp.log(sm)

    full = lambda s: pl.BlockSpec(s, lambda i: (0,) * len(s))
    rowb = pl.BlockSpec((BN, CH), lambda i: (i, 0))
    return pl.pallas_call(
        body,
        grid=grid,
        in_specs=[pl.BlockSpec((2, BN, CH), lambda i: (0, i, 0)), rowb,
                  full((CH, CH)), full((1, CH))],
        out_specs=rowb,
        out_shape=jax.ShapeDtypeStruct((NPAD, CH), jnp.float32),
    )(agg1, dis, W2p, b2p)


_NREAL = 10000


def kernel(x, edge_index, W1, b1, Wq0, bq0, Wk0, bk0, Wv0, bv0,
           Wq1, bq1, Wk1, bk1, Wv1, bv1, W2, b2):
    global _NREAL
    N, F = x.shape
    _NREAL = N
    E = edge_index.shape[1]
    nclass = W2.shape[1]
    idt = edge_index.dtype

    loops = jnp.arange(N, dtype=idt)
    row = jnp.concatenate([edge_index[0], loops])
    col = jnp.concatenate([edge_index[1], loops])
    etot = E + N
    gran = NW * BE * 2
    epad = ((etot + gran - 1) // gran) * gran
    padidx = jnp.full((epad - etot,), NPAD - 1, idt)
    row_p = jnp.concatenate([row, padidx]).astype(jnp.int32)
    col_p = jnp.concatenate([col, padidx]).astype(jnp.int32)

    x_p = jnp.pad(x, ((0, NPAD - N), (0, 0)))
    r2 = lambda b: b.reshape(1, CH)
    bd = _block_diag

    deg2 = _degree(col_p)
    h, q0, k0, v0p, dis = _tc1(x_p, W1, r2(b1), bd(Wq0), r2(bq0),
                               bd(Wk0), r2(bk0), bd(Wv0), r2(bv0), deg2)
    agg0 = _conv(1, row_p, col_p, q0, k0, v0p)
    q1, k1c, v1c = _tc2(h, agg0, dis, bd(Wq1), r2(bq1), bd(Wk1), r2(bk1),
                        bd(Wv1), r2(bv1))
    agg1 = _conv(2, row_p, col_p, q1, k1c, v1c)
    W2p = jnp.pad(W2, ((0, 0), (0, CH - nclass)))
    b2p = jnp.pad(b2, (0, CH - nclass)).reshape(1, CH)
    out = _tc3(agg1, dis, W2p, b2p, nclass)
    return out[:N, :nclass]
